# baseline (speedup 1.0000x reference)
import functools

import jax
import jax.numpy as jnp
from jax import lax
from jax.experimental import pallas as pl
from jax.experimental.pallas import tpu as pltpu
from jax.experimental.pallas import tpu_sc as plsc

N = 10000
E = 320000
NC = 2
NS = 16
NT = NC * NS
NPAD = 10240
EB = 125
NEB = (E // NT) // EB
NEBH = NEB // 2
RPT = NPAD // NS
BM = 1280
GRID = NPAD // BM


def _sc_degree():
    mesh = plsc.VectorSubcoreMesh(core_axis_name="c", subcore_axis_name="s")

    @functools.partial(
        pl.kernel, mesh=mesh,
        out_type=jax.ShapeDtypeStruct((NC, NPAD, 128), jnp.float32),
        scratch_types=[
            pltpu.VMEM((NEB, EB), jnp.int32),
            pltpu.VMEM((EB, 128), jnp.float32),
            pltpu.VMEM_SHARED((NPAD, 128), jnp.float32),
            pltpu.SemaphoreType.DMA,
        ],
    )
    def k(dst_hbm, ones_hbm, zeros_hbm, out_hbm, didx, ones_v, acc, sem):
        cid = lax.axis_index("c")
        sid = lax.axis_index("s")
        wid = cid * NS + sid
        pltpu.sync_copy(dst_hbm.at[pl.ds(wid * NEB, NEB)], didx)
        pltpu.sync_copy(ones_hbm, ones_v)
        pltpu.sync_copy(zeros_hbm.at[pl.ds(sid * RPT, RPT)],
                        acc.at[pl.ds(sid * RPT, RPT)])
        plsc.subcore_barrier()

        def body(j, carry):
            pltpu.async_copy(ones_v, acc.at[didx.at[j]], sem, add=True)
            return carry

        lax.fori_loop(0, NEB, body, 0)

        def drain(j, carry):
            pltpu.make_async_copy(ones_v, acc.at[didx.at[j]], sem).wait()
            return carry

        lax.fori_loop(0, NEB, drain, 0)
        plsc.subcore_barrier()
        pltpu.sync_copy(acc.at[pl.ds(sid * RPT, RPT)],
                        out_hbm.at[cid, pl.ds(sid * RPT, RPT)])

    return k


def _sc_agg(nchunks):
    mesh = plsc.VectorSubcoreMesh(core_axis_name="c", subcore_axis_name="s")

    @functools.partial(
        pl.kernel, mesh=mesh,
        out_type=[jax.ShapeDtypeStruct((NC, NPAD, 128), jnp.float32)
                  for _ in range(nchunks)],
        scratch_types=[
            pltpu.VMEM((NEBH, EB), jnp.int32),
            pltpu.VMEM((NEBH, EB), jnp.int32),
            pltpu.VMEM((EB, 128), jnp.float32),
            pltpu.VMEM((EB, 128), jnp.float32),
            pltpu.VMEM_SHARED((NPAD, 128), jnp.float32),
            pltpu.SemaphoreType.DMA,
            pltpu.SemaphoreType.DMA,
            pltpu.SemaphoreType.DMA,
            pltpu.SemaphoreType.DMA,
        ],
    )
    def k(src_hbm, dst_hbm, zeros_hbm, *rest):
        tabs = rest[:nchunks]
        outs = rest[nchunks:2 * nchunks]
        (sidx, didx, rows0, rows1, acc,
         sem0, sem1, sem2, sem3) = rest[2 * nchunks:]
        cid = lax.axis_index("c")
        sid = lax.axis_index("s")
        wid = cid * NS + sid
        half = NEBH // 2
        for c in range(nchunks):
            pltpu.sync_copy(zeros_hbm.at[pl.ds(sid * RPT, RPT)],
                            acc.at[pl.ds(sid * RPT, RPT)])
            plsc.subcore_barrier()

            for h in range(2):
                pltpu.sync_copy(
                    src_hbm.at[pl.ds(wid * NEB + h * NEBH, NEBH)], sidx)
                pltpu.sync_copy(
                    dst_hbm.at[pl.ds(wid * NEB + h * NEBH, NEBH)], didx)

                def prologue(t, carry):
                    pltpu.async_copy(tabs[c].at[sidx.at[2 * t]], rows0, sem0)
                    pltpu.async_copy(tabs[c].at[sidx.at[2 * t + 1]],
                                     rows1, sem1)
                    return carry

                lax.fori_loop(0, 1, prologue, 0)

                def body(jj, carry):
                    j = 2 * jj
                    pltpu.make_async_copy(tabs[c].at[sidx.at[j]],
                                          rows0, sem0).wait()
                    pltpu.async_copy(rows0, acc.at[didx.at[j]],
                                     sem2, add=True)
                    pltpu.make_async_copy(tabs[c].at[sidx.at[j + 1]],
                                          rows1, sem1).wait()
                    pltpu.async_copy(rows1, acc.at[didx.at[j + 1]],
                                     sem3, add=True)
                    pltpu.make_async_copy(rows0, acc.at[didx.at[j]],
                                          sem2).wait()
                    pltpu.async_copy(tabs[c].at[sidx.at[j + 2]], rows0, sem0)
                    pltpu.make_async_copy(rows1, acc.at[didx.at[j + 1]],
                                          sem3).wait()
                    pltpu.async_copy(tabs[c].at[sidx.at[j + 3]], rows1, sem1)
                    return carry

                lax.fori_loop(0, half - 1, body, 0)

                def epilogue(jj, carry):
                    j = 2 * jj
                    pltpu.make_async_copy(tabs[c].at[sidx.at[j]],
                                          rows0, sem0).wait()
                    pltpu.async_copy(rows0, acc.at[didx.at[j]],
                                     sem2, add=True)
                    pltpu.make_async_copy(tabs[c].at[sidx.at[j + 1]],
                                          rows1, sem1).wait()
                    pltpu.async_copy(rows1, acc.at[didx.at[j + 1]],
                                     sem3, add=True)
                    pltpu.make_async_copy(rows0, acc.at[didx.at[j]],
                                          sem2).wait()
                    pltpu.make_async_copy(rows1, acc.at[didx.at[j + 1]],
                                          sem3).wait()
                    return carry

                lax.fori_loop(half - 1, half, epilogue, 0)
            plsc.subcore_barrier()
            pltpu.sync_copy(acc.at[pl.ds(sid * RPT, RPT)],
                            outs[c].at[cid, pl.ds(sid * RPT, RPT)])
            plsc.subcore_barrier()

    return k


def _tc_prep(deg, v):
    def body(deg_ref, v_ref, dinv_ref, y0_ref):
        d = deg_ref[0, :, :1] + deg_ref[1, :, :1] + 1.0
        dinv = lax.rsqrt(d)
        dinv_ref[...] = jnp.broadcast_to(dinv, (BM, 16))
        y0_ref[...] = v_ref[...] * dinv

    return pl.pallas_call(
        body,
        grid=(GRID,),
        in_specs=[
            pl.BlockSpec((NC, BM, 128), lambda i: (0, i, 0)),
            pl.BlockSpec((BM, 128), lambda i: (i, 0)),
        ],
        out_specs=[
            pl.BlockSpec((BM, 16), lambda i: (i, 0)),
            pl.BlockSpec((BM, 128), lambda i: (i, 0)),
        ],
        out_shape=[
            jax.ShapeDtypeStruct((NPAD, 16), jnp.float32),
            jax.ShapeDtypeStruct((NPAD, 128), jnp.float32),
        ],
    )(deg, v)


def _tc_layer(achunks, ychunks, dinv, W, b, nout, relu_scale):
    nin = len(achunks)
    fout = W.shape[1]
    out_w = fout // nout

    def body(*refs):
        a_refs = refs[:nin]
        y_refs = refs[nin:2 * nin]
        dinv_ref, w_ref, b_ref = refs[2 * nin:2 * nin + 3]
        out_refs = refs[2 * nin + 3:]
        dinv = dinv_ref[:, :1]
        t = jnp.concatenate(
            [a[0] + a[1] + y[...] for a, y in zip(a_refs, y_refs)], axis=1)
        t = t * dinv
        z = jnp.dot(t, w_ref[...], preferred_element_type=jnp.float32,
                    precision=lax.Precision.HIGHEST) + b_ref[...]
        if relu_scale:
            z = jnp.maximum(z, 0.0) * dinv
        if nout == 1:
            out_refs[0][...] = z
        else:
            for c in range(nout):
                out_refs[c][...] = z[:, out_w * c:out_w * (c + 1)]

    fin = 128 * nin
    return pl.pallas_call(
        body,
        grid=(GRID,),
        in_specs=(
            [pl.BlockSpec((NC, BM, 128), lambda i: (0, i, 0))] * nin
            + [pl.BlockSpec((BM, 128), lambda i: (i, 0))] * nin
            + [pl.BlockSpec((BM, 16), lambda i: (i, 0)),
               pl.BlockSpec((fin, fout), lambda i: (0, 0)),
               pl.BlockSpec((1, fout), lambda i: (0, 0))]
        ),
        out_specs=[pl.BlockSpec((BM, out_w), lambda i: (i, 0))] * nout,
        out_shape=[jax.ShapeDtypeStruct((NPAD, out_w), jnp.float32)] * nout,
    )(*achunks, *ychunks, dinv, W, b)


def kernel(v, edge_index, W1, b1, W2, b2, W3, b3, Wmu, bmu, Wstd, bstd):
    src = edge_index[0].astype(jnp.int32).reshape(NT * NEB, EB)
    dst = edge_index[1].astype(jnp.int32).reshape(NT * NEB, EB)
    vp = jnp.zeros((NPAD, 128), jnp.float32).at[:N].set(v)
    zeros128 = jnp.zeros((NPAD, 128), jnp.float32)
    ones128 = jnp.ones((EB, 128), jnp.float32)

    deg = _sc_degree()(dst, ones128, zeros128)
    dinv, y0 = _tc_prep(deg, vp)

    (a1,) = _sc_agg(1)(src, dst, zeros128, y0)
    (y1,) = _tc_layer([a1], [y0], dinv, W1, b1.reshape(1, -1), 1, True)

    (a2,) = _sc_agg(1)(src, dst, zeros128, y1)
    y2 = _tc_layer([a2], [y1], dinv, W2, b2.reshape(1, -1), 2, True)

    a3 = _sc_agg(2)(src, dst, zeros128, *y2)
    y3 = _tc_layer(a3, y2, dinv, W3, b3.reshape(1, -1), 4, True)

    a4 = _sc_agg(4)(src, dst, zeros128, *y3)
    Wh = jnp.concatenate([Wmu, Wstd], axis=1)
    bh = jnp.concatenate([bmu, bstd]).reshape(1, -1)
    mu, std = _tc_layer(a4, y3, dinv, Wh, bh, 2, False)

    return (mu[:N], std[:N])


# speedup vs baseline: 1.2556x; 1.2556x over previous
import functools

import jax
import jax.numpy as jnp
from jax import lax
from jax.experimental import pallas as pl
from jax.experimental.pallas import tpu as pltpu
from jax.experimental.pallas import tpu_sc as plsc

N = 10000
E = 320000
NC = 2
NS = 16
NT = NC * NS
NPAD = 10240
EB = 125
NEB = (E // NT) // EB
NEBH = NEB // 2
RPT = NPAD // NS
BM = 1280
GRID = NPAD // BM


def _sc_degree():
    mesh = plsc.VectorSubcoreMesh(core_axis_name="c", subcore_axis_name="s")

    @functools.partial(
        pl.kernel, mesh=mesh,
        out_type=jax.ShapeDtypeStruct((NC, NPAD, 128), jnp.float32),
        scratch_types=[
            pltpu.VMEM((NEB, EB), jnp.int32),
            pltpu.VMEM((EB, 128), jnp.float32),
            pltpu.VMEM_SHARED((NPAD, 128), jnp.float32),
        ],
    )
    def k(dst_hbm, ones_hbm, zeros_hbm, out_hbm, didx, ones_v, acc):
        cid = lax.axis_index("c")
        sid = lax.axis_index("s")
        wid = cid * NS + sid
        pltpu.sync_copy(dst_hbm.at[pl.ds(wid * NEB, NEB)], didx)
        pltpu.sync_copy(ones_hbm, ones_v)
        pltpu.sync_copy(zeros_hbm.at[pl.ds(sid * RPT, RPT)],
                        acc.at[pl.ds(sid * RPT, RPT)])
        plsc.subcore_barrier()

        def body(j, carry):
            pltpu.sync_copy(ones_v, acc.at[didx.at[j]], add=True)
            return carry

        lax.fori_loop(0, NEB, body, 0)
        plsc.subcore_barrier()
        pltpu.sync_copy(acc.at[pl.ds(sid * RPT, RPT)],
                        out_hbm.at[cid, pl.ds(sid * RPT, RPT)])

    return k


def _sc_agg(nchunks):
    mesh = plsc.VectorSubcoreMesh(core_axis_name="c", subcore_axis_name="s")

    @functools.partial(
        pl.kernel, mesh=mesh,
        out_type=[jax.ShapeDtypeStruct((NC, NPAD, 128), jnp.float32)
                  for _ in range(nchunks)],
        scratch_types=[
            pltpu.VMEM((NEBH, EB), jnp.int32),
            pltpu.VMEM((NEBH, EB), jnp.int32),
            pltpu.VMEM((EB, 128), jnp.float32),
            pltpu.VMEM((EB, 128), jnp.float32),
            pltpu.VMEM_SHARED((NPAD, 128), jnp.float32),
            pltpu.SemaphoreType.DMA,
            pltpu.SemaphoreType.DMA,
        ],
    )
    def k(src_hbm, dst_hbm, zeros_hbm, *rest):
        tabs = rest[:nchunks]
        outs = rest[nchunks:2 * nchunks]
        sidx, didx, rows0, rows1, acc, sem0, sem1 = rest[2 * nchunks:]
        cid = lax.axis_index("c")
        sid = lax.axis_index("s")
        wid = cid * NS + sid
        half = NEBH // 2
        for c in range(nchunks):
            pltpu.sync_copy(zeros_hbm.at[pl.ds(sid * RPT, RPT)],
                            acc.at[pl.ds(sid * RPT, RPT)])
            plsc.subcore_barrier()

            for h in range(2):
                pltpu.sync_copy(
                    src_hbm.at[pl.ds(wid * NEB + h * NEBH, NEBH)], sidx)
                pltpu.sync_copy(
                    dst_hbm.at[pl.ds(wid * NEB + h * NEBH, NEBH)], didx)

                def prologue(t, carry):
                    pltpu.async_copy(tabs[c].at[sidx.at[2 * t]], rows0, sem0)
                    pltpu.async_copy(tabs[c].at[sidx.at[2 * t + 1]],
                                     rows1, sem1)
                    return carry

                lax.fori_loop(0, 1, prologue, 0)

                def body(jj, carry):
                    j = 2 * jj
                    pltpu.make_async_copy(tabs[c].at[sidx.at[j]],
                                          rows0, sem0).wait()
                    pltpu.sync_copy(rows0, acc.at[didx.at[j]], add=True)
                    pltpu.async_copy(tabs[c].at[sidx.at[j + 2]], rows0, sem0)
                    pltpu.make_async_copy(tabs[c].at[sidx.at[j + 1]],
                                          rows1, sem1).wait()
                    pltpu.sync_copy(rows1, acc.at[didx.at[j + 1]], add=True)
                    pltpu.async_copy(tabs[c].at[sidx.at[j + 3]], rows1, sem1)
                    return carry

                lax.fori_loop(0, half - 1, body, 0)

                def epilogue(jj, carry):
                    j = 2 * jj
                    pltpu.make_async_copy(tabs[c].at[sidx.at[j]],
                                          rows0, sem0).wait()
                    pltpu.sync_copy(rows0, acc.at[didx.at[j]], add=True)
                    pltpu.make_async_copy(tabs[c].at[sidx.at[j + 1]],
                                          rows1, sem1).wait()
                    pltpu.sync_copy(rows1, acc.at[didx.at[j + 1]], add=True)
                    return carry

                lax.fori_loop(half - 1, half, epilogue, 0)
            plsc.subcore_barrier()
            pltpu.sync_copy(acc.at[pl.ds(sid * RPT, RPT)],
                            outs[c].at[cid, pl.ds(sid * RPT, RPT)])
            plsc.subcore_barrier()

    return k


def _tc_prep(deg, v):
    def body(deg_ref, v_ref, dinv_ref, y0_ref):
        d = deg_ref[0, :, :1] + deg_ref[1, :, :1] + 1.0
        dinv = lax.rsqrt(d)
        dinv_ref[...] = jnp.broadcast_to(dinv, (BM, 16))
        y0_ref[...] = v_ref[...] * dinv

    return pl.pallas_call(
        body,
        grid=(GRID,),
        in_specs=[
            pl.BlockSpec((NC, BM, 128), lambda i: (0, i, 0)),
            pl.BlockSpec((BM, 128), lambda i: (i, 0)),
        ],
        out_specs=[
            pl.BlockSpec((BM, 16), lambda i: (i, 0)),
            pl.BlockSpec((BM, 128), lambda i: (i, 0)),
        ],
        out_shape=[
            jax.ShapeDtypeStruct((NPAD, 16), jnp.float32),
            jax.ShapeDtypeStruct((NPAD, 128), jnp.float32),
        ],
    )(deg, v)


def _tc_layer(achunks, ychunks, dinv, W, b, nout, relu_scale):
    nin = len(achunks)
    fout = W.shape[1]
    out_w = fout // nout

    def body(*refs):
        a_refs = refs[:nin]
        y_refs = refs[nin:2 * nin]
        dinv_ref, w_ref, b_ref = refs[2 * nin:2 * nin + 3]
        out_refs = refs[2 * nin + 3:]
        dinv = dinv_ref[:, :1]
        t = jnp.concatenate(
            [a[0] + a[1] + y[...] for a, y in zip(a_refs, y_refs)], axis=1)
        t = t * dinv
        z = jnp.dot(t, w_ref[...], preferred_element_type=jnp.float32,
                    precision=lax.Precision.HIGHEST) + b_ref[...]
        if relu_scale:
            z = jnp.maximum(z, 0.0) * dinv
        if nout == 1:
            out_refs[0][...] = z
        else:
            for c in range(nout):
                out_refs[c][...] = z[:, out_w * c:out_w * (c + 1)]

    fin = 128 * nin
    return pl.pallas_call(
        body,
        grid=(GRID,),
        in_specs=(
            [pl.BlockSpec((NC, BM, 128), lambda i: (0, i, 0))] * nin
            + [pl.BlockSpec((BM, 128), lambda i: (i, 0))] * nin
            + [pl.BlockSpec((BM, 16), lambda i: (i, 0)),
               pl.BlockSpec((fin, fout), lambda i: (0, 0)),
               pl.BlockSpec((1, fout), lambda i: (0, 0))]
        ),
        out_specs=[pl.BlockSpec((BM, out_w), lambda i: (i, 0))] * nout,
        out_shape=[jax.ShapeDtypeStruct((NPAD, out_w), jnp.float32)] * nout,
    )(*achunks, *ychunks, dinv, W, b)


def kernel(v, edge_index, W1, b1, W2, b2, W3, b3, Wmu, bmu, Wstd, bstd):
    src = edge_index[0].astype(jnp.int32).reshape(NT * NEB, EB)
    dst = edge_index[1].astype(jnp.int32).reshape(NT * NEB, EB)
    vp = jnp.zeros((NPAD, 128), jnp.float32).at[:N].set(v)
    zeros128 = jnp.zeros((NPAD, 128), jnp.float32)
    ones128 = jnp.ones((EB, 128), jnp.float32)

    deg = _sc_degree()(dst, ones128, zeros128)
    dinv, y0 = _tc_prep(deg, vp)

    (a1,) = _sc_agg(1)(src, dst, zeros128, y0)
    (y1,) = _tc_layer([a1], [y0], dinv, W1, b1.reshape(1, -1), 1, True)

    (a2,) = _sc_agg(1)(src, dst, zeros128, y1)
    y2 = _tc_layer([a2], [y1], dinv, W2, b2.reshape(1, -1), 2, True)

    a3 = _sc_agg(2)(src, dst, zeros128, *y2)
    y3 = _tc_layer(a3, y2, dinv, W3, b3.reshape(1, -1), 4, True)

    a4 = _sc_agg(4)(src, dst, zeros128, *y3)
    Wh = jnp.concatenate([Wmu, Wstd], axis=1)
    bh = jnp.concatenate([bmu, bstd]).reshape(1, -1)
    mu, std = _tc_layer(a4, y3, dinv, Wh, bh, 2, False)

    return (mu[:N], std[:N])
